# baseline (speedup 1.0000x reference)
import jax
import jax.numpy as jnp
from jax.experimental import pallas as pl
from jax.experimental.pallas import tpu as pltpu

S_BLK = 256


def _body(lens_ref, x_ref, w_ref, o_ref, ml_ref, acc_ref):
    b = pl.program_id(0)
    j = pl.program_id(1)
    length = lens_ref[b]
    last = (length - 1) // S_BLK

    @pl.when(j == 0)
    def _init():
        ml_ref[0] = -jnp.inf
        ml_ref[1] = 0.0
        acc_ref[...] = jnp.zeros_like(acc_ref)

    @pl.when(j <= last)
    def _compute():
        xb = x_ref[0]
        logits = jax.lax.dot_general(
            xb, w_ref[...], (((1,), (0,)), ((), ())),
            preferred_element_type=jnp.float32)
        pos = j * S_BLK + jax.lax.broadcasted_iota(jnp.int32, (S_BLK, 1), 0)
        mask = pos < length
        logits = jnp.where(mask, logits, -jnp.inf)
        m_prev = ml_ref[0]
        m_new = jnp.maximum(m_prev, jnp.max(logits))
        alpha = jnp.exp(m_prev - m_new)
        p = jnp.where(mask, jnp.exp(logits - m_new), 0.0)
        ml_ref[0] = m_new
        ml_ref[1] = ml_ref[1] * alpha + jnp.sum(p)
        px = jax.lax.dot_general(
            p, xb, (((0,), (0,)), ((), ())),
            preferred_element_type=jnp.float32)
        acc_ref[...] = acc_ref[...] * alpha + px

        @pl.when(j == last)
        def _fin():
            o_ref[0] = acc_ref[...] / ml_ref[1]


def kernel(x, x_lens, W, b):
    B, S, D = x.shape
    lens = x_lens.astype(jnp.int32)
    grid = (B, S // S_BLK)
    return pl.pallas_call(
        _body,
        grid_spec=pltpu.PrefetchScalarGridSpec(
            num_scalar_prefetch=1,
            grid=grid,
            in_specs=[
                pl.BlockSpec(
                    (1, S_BLK, D),
                    lambda bi, j, lens: (bi, jnp.minimum(j, (lens[bi] - 1) // S_BLK), 0)),
                pl.BlockSpec((D, 1), lambda bi, j, lens: (0, 0)),
            ],
            out_specs=pl.BlockSpec((1, 1, D), lambda bi, j, lens: (bi, 0, 0)),
            scratch_shapes=[
                pltpu.SMEM((2,), jnp.float32),
                pltpu.VMEM((1, D), jnp.float32),
            ],
        ),
        out_shape=jax.ShapeDtypeStruct((B, 1, D), jnp.float32),
        compiler_params=pltpu.CompilerParams(
            dimension_semantics=("arbitrary", "arbitrary")),
    )(lens, x, W)[:, 0, :]


# speedup vs baseline: 1.6432x; 1.6432x over previous
import jax
import jax.numpy as jnp
from jax.experimental import pallas as pl
from jax.experimental.pallas import tpu as pltpu

S_BLK = 256


def _body(x_ref, lens_ref, w_ref, o_ref, m_ref, l_ref, acc_ref):
    j = pl.program_id(0)
    nsteps = pl.num_programs(0)
    B, _, D = x_ref.shape

    @pl.when(j == 0)
    def _init():
        m_ref[...] = jnp.full_like(m_ref, -jnp.inf)
        l_ref[...] = jnp.zeros_like(l_ref)
        acc_ref[...] = jnp.zeros_like(acc_ref)

    xb = x_ref[...]
    xflat = xb.reshape(B * S_BLK, D)
    logits_flat = jax.lax.dot_general(
        xflat, w_ref[...], (((1,), (0,)), ((), ())),
        preferred_element_type=jnp.float32)
    logits = logits_flat.reshape(B, S_BLK, 1)
    pos = j * S_BLK + jax.lax.broadcasted_iota(jnp.int32, (B, S_BLK, 1), 1)
    mask = pos < lens_ref[...][:, :, None]
    logits = jnp.where(mask, logits, -jnp.inf)
    m_prev = m_ref[...]
    m_new = jnp.maximum(m_prev, jnp.max(logits, axis=1))
    alpha = jnp.exp(m_prev - m_new)
    alpha = jnp.where(m_new == -jnp.inf, 0.0, alpha)
    p = jnp.where(mask, jnp.exp(logits - m_new[:, :, None]), 0.0)
    l_ref[...] = l_ref[...] * alpha + jnp.sum(p, axis=1)
    m_ref[...] = m_new
    p2 = p.reshape(B, S_BLK)
    px = jax.lax.dot_general(
        p2, xb, (((1,), (1,)), ((0,), (0,))),
        preferred_element_type=jnp.float32)
    acc_ref[...] = acc_ref[...] * alpha + px

    @pl.when(j == nsteps - 1)
    def _fin():
        o_ref[...] = acc_ref[...] / l_ref[...]


def kernel(x, x_lens, W, b):
    B, S, D = x.shape
    lens2 = x_lens.astype(jnp.int32).reshape(B, 1)
    grid = (S // S_BLK,)
    return pl.pallas_call(
        _body,
        grid=grid,
        in_specs=[
            pl.BlockSpec((B, S_BLK, D), lambda j: (0, j, 0)),
            pl.BlockSpec((B, 1), lambda j: (0, 0)),
            pl.BlockSpec((D, 1), lambda j: (0, 0)),
        ],
        out_specs=pl.BlockSpec((B, D), lambda j: (0, 0)),
        scratch_shapes=[
            pltpu.VMEM((B, 1), jnp.float32),
            pltpu.VMEM((B, 1), jnp.float32),
            pltpu.VMEM((B, D), jnp.float32),
        ],
        out_shape=jax.ShapeDtypeStruct((B, D), jnp.float32),
        compiler_params=pltpu.CompilerParams(
            dimension_semantics=("arbitrary",)),
    )(x, lens2, W)
